# baseline (speedup 1.0000x reference)
import functools

import jax
import jax.numpy as jnp
from jax import lax
from jax.experimental import pallas as pl
from jax.experimental.pallas import tpu as pltpu
from jax.experimental.pallas import tpu_sc as plsc

_NUM_WORKERS = 32
_CHUNK = 128
_GROUP = 16


def _make_sc_kernel(B, D):
    rows_per_w = B // _NUM_WORKERS
    n_chunks = rows_per_w // _CHUNK
    n_groups = rows_per_w // _GROUP
    n_dim_chunks = D // 16

    mesh = plsc.VectorSubcoreMesh(core_axis_name="c", subcore_axis_name="s")

    @functools.partial(
        pl.kernel,
        out_type=jax.ShapeDtypeStruct((B,), jnp.float32),
        mesh=mesh,
        compiler_params=pltpu.CompilerParams(use_tc_tiling_on_sc=False),
        scratch_types=[
            pltpu.VMEM((n_chunks, _CHUNK), jnp.int32),
            pltpu.VMEM((n_chunks, _CHUNK), jnp.int32),
            pltpu.VMEM((rows_per_w, D), jnp.float32),
            pltpu.VMEM((rows_per_w, D), jnp.float32),
            pltpu.VMEM((rows_per_w,), jnp.float32),
            pltpu.VMEM((rows_per_w,), jnp.float32),
            pltpu.VMEM((16,), jnp.float32),
            pltpu.VMEM((3 * 512,), jnp.float32),
            pltpu.VMEM((rows_per_w,), jnp.float32),
            pltpu.SemaphoreType.DMA,
        ],
    )
    def svd_kernel(uidx_hbm, iidx_hbm, uembd_hbm, iembd_hbm, ubias_hbm,
                   ibias_hbm, oab_hbm, out_hbm, uidx_v, iidx_v, urows_v,
                   irows_v, ubias_v, ibias_v, oab_v, fb_v, out_v, sem):
        wid = lax.axis_index("s") * 2 + lax.axis_index("c")

        pltpu.sync_copy(uidx_hbm.at[wid], uidx_v)
        pltpu.sync_copy(iidx_hbm.at[wid], iidx_v)
        pltpu.sync_copy(oab_hbm, oab_v.at[pl.ds(0, 1)])

        copies = []
        for j in range(n_chunks):
            rows = pl.ds(j * _CHUNK, _CHUNK)
            copies.append(pltpu.async_copy(
                uembd_hbm.at[uidx_v.at[j]], urows_v.at[rows], sem))
            copies.append(pltpu.async_copy(
                iembd_hbm.at[iidx_v.at[j]], irows_v.at[rows], sem))
            copies.append(pltpu.async_copy(
                ubias_hbm.at[uidx_v.at[j]], ubias_v.at[rows], sem))
            copies.append(pltpu.async_copy(
                ibias_hbm.at[iidx_v.at[j]], ibias_v.at[rows], sem))
        for c in copies:
            c.wait()

        iota16 = lax.iota(jnp.int32, 16)
        oab = oab_v[pl.ds(0, 16)][0]

        def group_body(g, carry):
            base = g * _GROUP
            res = ubias_v[pl.ds(base, 16)] + ibias_v[pl.ds(base, 16)] + oab
            dots = res * 0.0
            for r in range(_GROUP):
                row = base + r
                acc = urows_v[row, pl.ds(0, 16)] * irows_v[row, pl.ds(0, 16)]
                for cdim in range(1, n_dim_chunks):
                    sl = pl.ds(cdim * 16, 16)
                    acc += urows_v[row, sl] * irows_v[row, sl]
                f1 = acc + lax.rev(acc, (0,))
                fb_v[pl.ds(32 * r, 16)] = f1
                f2 = f1 + fb_v[pl.ds(32 * r + 4, 16)]
                fb_v[pl.ds(512 + 32 * r, 16)] = f2
                f3 = f2 + fb_v[pl.ds(512 + 32 * r + 2, 16)]
                fb_v[pl.ds(1024 + 32 * r, 16)] = f3
                f4 = f3 + fb_v[pl.ds(1024 + 32 * r + 1, 16)]
                dots = jnp.where(iota16 == r, f4[0], dots)
            out_v[pl.ds(base, 16)] = dots + res
            return carry

        lax.fori_loop(0, n_groups, group_body, 0)

        pltpu.sync_copy(out_v, out_hbm.at[pl.ds(wid * rows_per_w, rows_per_w)])

    return svd_kernel


@jax.jit
def kernel(userIdx, itemIdx, uEmbd, iEmbd, uBias, iBias, overAllBias):
    B = userIdx.shape[0]
    D = uEmbd.shape[1]
    uidx = userIdx.astype(jnp.int32).reshape(_NUM_WORKERS, -1, _CHUNK)
    iidx = itemIdx.astype(jnp.int32).reshape(_NUM_WORKERS, -1, _CHUNK)
    sc = _make_sc_kernel(B, D)
    return sc(uidx, iidx, uEmbd, iEmbd, uBias.reshape(-1), iBias.reshape(-1),
              overAllBias.astype(jnp.float32))
